# baseline (speedup 1.0000x reference)
import functools

import jax
import jax.numpy as jnp
from jax import lax
from jax.experimental import pallas as pl
from jax.experimental.pallas import tpu as pltpu
from jax.experimental.pallas import tpu_sc as plsc

_DIM = 64
_B = 16384
_L = 20
_NC = 2
_NS = 16
_NW = _NC * _NS
_BPW = _B // _NW
_PPW = _BPW * _L
_CHUNKS = 16
_CB = _BPW // _CHUNKS
_CP = _CB * _L
_IDXW = 128
_QPC = _CP // _IDXW


def _sc_body(tgt_idx_hbm, ctx_idx_hbm, tt_hbm, ct_hbm, out_hbm,
             tgt_idx_v, ctx_idx_v, h_buf, u_buf, part, out_v, sem):
    wid = lax.axis_index("s") * _NC + lax.axis_index("c")
    pltpu.sync_copy(tgt_idx_hbm.at[wid], tgt_idx_v)
    pltpu.sync_copy(ctx_idx_hbm.at[wid], ctx_idx_v)
    iota = lax.iota(jnp.int32, 16)

    def chunk(k, carry):
        pltpu.async_copy(tt_hbm.at[tgt_idx_v.at[k]], h_buf, sem).wait()
        for q in range(_QPC):
            pltpu.async_copy(ct_hbm.at[ctx_idx_v.at[_QPC * k + q]],
                             u_buf.at[pl.ds(_IDXW * q, _IDXW)], sem).wait()

        def sub(s, c2):
            for bi in range(4):
                b = 4 * s + bi
                hs = [h_buf[b, pl.ds(16 * c, 16)] for c in range(4)]
                for l in range(_L):
                    r = 80 * s + 20 * bi + l
                    acc = u_buf[r, pl.ds(0, 16)] * hs[0]
                    for c in range(1, 4):
                        acc = acc + u_buf[r, pl.ds(16 * c, 16)] * hs[c]
                    part[20 * bi + l, :] = acc
            for t in range(5):
                rows = iota + 16 * t
                tot = plsc.load_gather(part, [rows, jnp.zeros(16, jnp.int32)])
                for c in range(1, 16):
                    tot = tot + plsc.load_gather(
                        part, [rows, jnp.full(16, c, jnp.int32)])
                sig = 1.0 / (1.0 + jnp.exp(-tot))
                out_v[pl.ds(_CP * k + 80 * s + 16 * t, 16)] = sig
            return c2

        return lax.fori_loop(0, _CB // 4, sub, carry)

    lax.fori_loop(0, _CHUNKS, chunk, 0)
    pltpu.sync_copy(out_v, out_hbm.at[wid])


@jax.jit
def _sc_call(tgt_idx3, ctx_idx3, target_table, context_table):
    mesh = plsc.VectorSubcoreMesh(core_axis_name="c", subcore_axis_name="s")
    return pl.kernel(
        _sc_body,
        out_type=jax.ShapeDtypeStruct((_NW, _PPW), jnp.float32),
        mesh=mesh,
        compiler_params=pltpu.CompilerParams(
            needs_layout_passes=False, use_tc_tiling_on_sc=False),
        scratch_types=[
            pltpu.VMEM((_CHUNKS, _CB), jnp.int32),
            pltpu.VMEM((_QPC * _CHUNKS, _IDXW), jnp.int32),
            pltpu.VMEM((_CB, _DIM), jnp.float32),
            pltpu.VMEM((_CP, _DIM), jnp.float32),
            pltpu.VMEM((80, 16), jnp.float32),
            pltpu.VMEM((_PPW,), jnp.float32),
            pltpu.SemaphoreType.DMA,
        ],
    )(tgt_idx3, ctx_idx3, target_table, context_table)


def kernel(target_word_id, context_word_ids, target_table, context_table):
    tgt3 = target_word_id.reshape(_NW, _CHUNKS, _CB)
    ctx3 = context_word_ids.reshape(_NW, _QPC * _CHUNKS, _IDXW)
    out = _sc_call(tgt3, ctx3, target_table, context_table)
    return out.reshape(_B, _L)


# speedup vs baseline: 1.0653x; 1.0653x over previous
import jax
import jax.numpy as jnp
from jax import lax
from jax.experimental import pallas as pl
from jax.experimental.pallas import tpu as pltpu
from jax.experimental.pallas import tpu_sc as plsc

_DIM = 64
_B = 16384
_L = 20
_NC = 2
_NS = 16
_NW = _NC * _NS
_BPW = _B // _NW
_PPW = _BPW * _L
_CHUNKS = 16
_CB = _BPW // _CHUNKS
_CP = _CB * _L
_IDXW = 128
_QPC = _CP // _IDXW


def _tree_sum(vs):
    while len(vs) > 1:
        vs = [vs[i] + vs[i + 1] for i in range(0, len(vs) - 1, 2)] + (
            [vs[-1]] if len(vs) % 2 else [])
    return vs[0]


def _sc_body(tgt_idx_hbm, ctx_idx_hbm, tt_hbm, ct_hbm, out_hbm,
             tgt_idx_v, ctx_idx_v, h_a, h_b, u_a, u_b, part, out_v,
             sem_a, sem_b):
    wid = lax.axis_index("s") * _NC + lax.axis_index("c")
    pltpu.sync_copy(tgt_idx_hbm.at[wid], tgt_idx_v)
    pltpu.sync_copy(ctx_idx_hbm.at[wid], ctx_idx_v)
    iota = lax.iota(jnp.int32, 16)

    def issue(k, h_buf, u_buf, sem):
        pltpu.async_copy(tt_hbm.at[tgt_idx_v.at[k]], h_buf, sem)
        for q in range(_QPC):
            pltpu.async_copy(ct_hbm.at[ctx_idx_v.at[_QPC * k + q]],
                             u_buf.at[pl.ds(_IDXW * q, _IDXW)], sem)

    def drain(k, h_buf, u_buf, sem):
        pltpu.make_async_copy(tt_hbm.at[tgt_idx_v.at[k]], h_buf, sem).wait()
        for q in range(_QPC):
            pltpu.make_async_copy(ct_hbm.at[ctx_idx_v.at[_QPC * k + q]],
                                  u_buf.at[pl.ds(_IDXW * q, _IDXW)],
                                  sem).wait()

    def compute(k, h_buf, u_buf):
        def sub(s, c2):
            hs = [[h_buf[4 * s + bi, pl.ds(16 * c, 16)] for c in range(4)]
                  for bi in range(4)]
            for t in range(5):
                for jj in range(16):
                    q = 16 * t + jj
                    h4 = hs[q // 20]
                    r = 80 * s + q
                    m0 = u_buf[r, pl.ds(0, 16)] * h4[0]
                    m1 = u_buf[r, pl.ds(16, 16)] * h4[1]
                    m2 = u_buf[r, pl.ds(32, 16)] * h4[2]
                    m3 = u_buf[r, pl.ds(48, 16)] * h4[3]
                    part[jj, :] = (m0 + m1) + (m2 + m3)
                cols = [plsc.load_gather(part, [iota, jnp.full(16, c, jnp.int32)])
                        for c in range(16)]
                tot = _tree_sum(cols)
                sig = 1.0 / (1.0 + jnp.exp(-tot))
                out_v[pl.ds(_CP * k + 80 * s + 16 * t, 16)] = sig
            return c2

        lax.fori_loop(0, _CB * _L // 80, sub, 0)

    issue(0, h_a, u_a, sem_a)

    def outer(i, carry):
        k = 2 * i
        issue(k + 1, h_b, u_b, sem_b)
        drain(k, h_a, u_a, sem_a)
        compute(k, h_a, u_a)

        @pl.when(i < _CHUNKS // 2 - 1)
        def _():
            issue(k + 2, h_a, u_a, sem_a)

        drain(k + 1, h_b, u_b, sem_b)
        compute(k + 1, h_b, u_b)
        return carry

    lax.fori_loop(0, _CHUNKS // 2, outer, 0)
    pltpu.sync_copy(out_v, out_hbm.at[wid])


@jax.jit
def _sc_call(tgt_idx3, ctx_idx3, target_table, context_table):
    mesh = plsc.VectorSubcoreMesh(core_axis_name="c", subcore_axis_name="s")
    return pl.kernel(
        _sc_body,
        out_type=jax.ShapeDtypeStruct((_NW, _PPW), jnp.float32),
        mesh=mesh,
        compiler_params=pltpu.CompilerParams(
            needs_layout_passes=False, use_tc_tiling_on_sc=False),
        scratch_types=[
            pltpu.VMEM((_CHUNKS, _CB), jnp.int32),
            pltpu.VMEM((_QPC * _CHUNKS, _IDXW), jnp.int32),
            pltpu.VMEM((_CB, _DIM), jnp.float32),
            pltpu.VMEM((_CB, _DIM), jnp.float32),
            pltpu.VMEM((_CP, _DIM), jnp.float32),
            pltpu.VMEM((_CP, _DIM), jnp.float32),
            pltpu.VMEM((16, 16), jnp.float32),
            pltpu.VMEM((_PPW,), jnp.float32),
            pltpu.SemaphoreType.DMA,
            pltpu.SemaphoreType.DMA,
        ],
    )(tgt_idx3, ctx_idx3, target_table, context_table)


def kernel(target_word_id, context_word_ids, target_table, context_table):
    tgt3 = target_word_id.reshape(_NW, _CHUNKS, _CB)
    ctx3 = context_word_ids.reshape(_NW, _QPC * _CHUNKS, _IDXW)
    out = _sc_call(tgt3, ctx3, target_table, context_table)
    return out.reshape(_B, _L)
